# baseline (speedup 1.0000x reference)
import functools

import jax
import jax.numpy as jnp
from jax.experimental import pallas as pl

N_POINTS = 2048
N_SAMPLES = 16
N_LAYER = 8


def _pdist2squared(x, y):
    xx = jnp.sum(x ** 2, axis=1)[:, :, None]
    yy = jnp.sum(y ** 2, axis=1)[:, None, :]
    d = xx + yy - 2.0 * jnp.einsum('bcn,bcp->bnp', x, y)
    d = jnp.where(jnp.isnan(d), 0.0, d)
    return jnp.clip(d, 0.0, None)


def _group_gather(feat, ind):
    return jax.vmap(lambda f, i: f[:, i])(feat, ind)


def _fps(xyz, npoint):
    B, N, _ = xyz.shape

    def body(i, state):
        idx, dists, far = state
        idx = idx.at[:, i].set(far)
        centroid = jnp.take_along_axis(xyz, far[:, None, None], axis=1)
        d = jnp.sum((xyz - centroid) ** 2, axis=-1)
        dists = jnp.minimum(dists, d)
        far = jnp.argmax(dists, axis=-1).astype(jnp.int32)
        return (idx, dists, far)

    init = (jnp.zeros((B, npoint), jnp.int32), jnp.full((B, N), 1e10, jnp.float32), jnp.zeros((B,), jnp.int32))
    idx, _, _ = jax.lax.fori_loop(0, npoint, body, init)
    return idx


def _bn_train(x, g, b, axes, eps=1e-3):
    m = jnp.mean(x, axis=axes, keepdims=True)
    v = jnp.var(x, axis=axes, keepdims=True)
    shape = [1] * x.ndim
    shape[1] = -1
    return (x - m) / jnp.sqrt(v + eps) * g.reshape(shape) + b.reshape(shape)


def _sample_k(xyz_full, xyz_sampled, num_samples):
    dist = _pdist2squared(xyz_full, xyz_sampled)
    _, ind = jax.lax.top_k(-jnp.transpose(dist, (0, 2, 1)), num_samples)
    return ind


def _pcconv(xyz_grouped, feat, ind, layer_params):
    feat_g = _group_gather(feat, ind)
    w = xyz_grouped
    for p in layer_params:
        w = jnp.einsum('bcps,oc->bops', w, p['W']) + p['b'][None, :, None, None]
        w = _bn_train(w, p['g'], p['be'], (0, 2, 3))
        w = jax.nn.relu(w)
    return jnp.einsum('bcps,bwps->bwp', feat_g, w) / w.shape[3]


def _three_nn(unknown, known):
    d = jnp.sum(unknown ** 2, -1)[:, :, None] + jnp.sum(known ** 2, -1)[:, None, :] - 2.0 * jnp.einsum('bnc,bpc->bnp', unknown, known)
    d = jnp.clip(d, 0.0, None)
    negd, ind = jax.lax.top_k(-d, 3)
    return jax.lax.stop_gradient(-negd), ind


def _linear_block(x, lin_params):
    for p in lin_params:
        x = x @ p['W'].T + p['b']
        x = _bn_train(x, p['g'], p['be'], (0,))
        x = jax.nn.relu(x)
    return x


def _feature_prop(xyz1, xyz2, feat1, feat2, fp_params):
    dist2, ind = _three_nn(jnp.transpose(xyz2, (0, 2, 1)), jnp.transpose(xyz1, (0, 2, 1)))
    inv = 1.0 / (dist2 + 1e-10)
    weights = inv / jnp.sum(inv, axis=2, keepdims=True)
    new = jnp.sum(_group_gather(feat1, ind) * weights[:, None, :, :], axis=3)
    new = jnp.concatenate([new, feat2], axis=1)
    x = new[..., None]
    for p in fp_params:
        x = jnp.einsum('bcns,oc->bons', x, p['W']) + p['b'][None, :, None, None]
        x = _bn_train(x, p['g'], p['be'], (0, 2, 3))
        x = jax.nn.relu(x)
    return x[..., 0]


def _final_conv_body(x_ref, w_ref, b_ref, o_ref):
    x = x_ref[0]
    w = w_ref[...]
    o_ref[0] = jnp.dot(w, x, preferred_element_type=jnp.float32) + b_ref[...][:, None]


def _final_conv(x, W, b):
    B, C, N = x.shape
    O = W.shape[0]
    return pl.pallas_call(
        _final_conv_body,
        grid=(B,),
        in_specs=[
            pl.BlockSpec((1, C, N), lambda i: (i, 0, 0)),
            pl.BlockSpec((O, C), lambda i: (0, 0)),
            pl.BlockSpec((O,), lambda i: (0,)),
        ],
        out_specs=pl.BlockSpec((1, O, N), lambda i: (i, 0, 0)),
        out_shape=jax.ShapeDtypeStruct((B, O, N), jnp.float32),
    )(x, W, b)


def _classifier_head(x, cls_params):
    for p in cls_params[:-1]:
        x = jnp.einsum('bcn,oc->bon', x, p['W']) + p['b'][None, :, None]
        x = _bn_train(x, p['g'], p['be'], (0, 2))
        x = jax.nn.relu(x)
    p = cls_params[-1]
    return _final_conv(x, p['W'], p['b'])


def kernel(xyz1, xyz2, feat1, feat2, params):
    xyz1_ind = _fps(jax.lax.stop_gradient(jnp.transpose(xyz1, (0, 2, 1))), N_POINTS)
    xyz1_1 = jax.vmap(lambda p, i: p[:, i])(xyz1, xyz1_ind)
    ind1 = _sample_k(xyz1, xyz1_1, N_SAMPLES)
    ind2 = _sample_k(xyz2, xyz1_1, N_SAMPLES)
    xyz1_g = _group_gather(xyz1, ind1) - xyz1_1[..., None]
    xyz2_g = _group_gather(xyz2, ind2) - xyz1_1[..., None]
    pc = params['pcconv']
    f1 = _pcconv(xyz1_g, feat1, ind1, pc[0])
    f2 = _pcconv(xyz2_g, feat2, ind2, pc[0])
    for i in range(1, N_LAYER - 1):
        f1 = f1 + _pcconv(xyz1_g, f1, ind1, pc[i])
        f2 = f2 + _pcconv(xyz2_g, f2, ind2, pc[i])
    f1 = _pcconv(xyz1_g, f1, ind1, pc[N_LAYER - 1])
    f2 = _pcconv(xyz2_g, f2, ind2, pc[N_LAYER - 1])
    p1 = _linear_block(jnp.max(f1, axis=2), params['lin'])[:, :, None]
    p2 = _linear_block(jnp.max(f2, axis=2), params['lin'])[:, :, None]
    P = f1.shape[-1]
    feat_final = jnp.concatenate([jnp.repeat(p1, P, axis=2), f1, jnp.repeat(p2, P, axis=2), f2], axis=1)
    feat_final = _feature_prop(xyz1_1, xyz1, feat_final, feat1, params['fp'])
    return _classifier_head(feat_final, params['cls'])


# speedup vs baseline: 1.2244x; 1.2244x over previous
import functools

import jax
import jax.numpy as jnp
from jax.experimental import pallas as pl
from jax.experimental.pallas import tpu as pltpu

N_POINTS = 2048
N_SAMPLES = 16
N_LAYER = 8


def _pdist2squared(x, y):
    xx = jnp.sum(x ** 2, axis=1)[:, :, None]
    yy = jnp.sum(y ** 2, axis=1)[:, None, :]
    d = xx + yy - 2.0 * jnp.einsum('bcn,bcp->bnp', x, y)
    d = jnp.where(jnp.isnan(d), 0.0, d)
    return jnp.clip(d, 0.0, None)


def _group_gather(feat, ind):
    return jax.vmap(lambda f, i: f[:, i])(feat, ind)


def _fps(xyz, npoint):
    B, N, _ = xyz.shape

    def body(i, state):
        idx, dists, far = state
        idx = idx.at[:, i].set(far)
        centroid = jnp.take_along_axis(xyz, far[:, None, None], axis=1)
        d = jnp.sum((xyz - centroid) ** 2, axis=-1)
        dists = jnp.minimum(dists, d)
        far = jnp.argmax(dists, axis=-1).astype(jnp.int32)
        return (idx, dists, far)

    init = (jnp.zeros((B, npoint), jnp.int32), jnp.full((B, N), 1e10, jnp.float32), jnp.zeros((B,), jnp.int32))
    idx, _, _ = jax.lax.fori_loop(0, npoint, body, init)
    return idx


def _bn_train(x, g, b, axes, eps=1e-3):
    m = jnp.mean(x, axis=axes, keepdims=True)
    v = jnp.var(x, axis=axes, keepdims=True)
    shape = [1] * x.ndim
    shape[1] = -1
    return (x - m) / jnp.sqrt(v + eps) * g.reshape(shape) + b.reshape(shape)


def _sample_k(xyz_full, xyz_sampled, num_samples):
    dist = _pdist2squared(xyz_full, xyz_sampled)
    _, ind = jax.lax.top_k(-jnp.transpose(dist, (0, 2, 1)), num_samples)
    return ind


def _pcconv(xyz_grouped, feat, ind, layer_params):
    feat_g = _group_gather(feat, ind)
    w = xyz_grouped
    for p in layer_params:
        w = jnp.einsum('bcps,oc->bops', w, p['W']) + p['b'][None, :, None, None]
        w = _bn_train(w, p['g'], p['be'], (0, 2, 3))
        w = jax.nn.relu(w)
    return jnp.einsum('bcps,bwps->bwp', feat_g, w) / w.shape[3]


def _three_nn(unknown, known):
    d = jnp.sum(unknown ** 2, -1)[:, :, None] + jnp.sum(known ** 2, -1)[:, None, :] - 2.0 * jnp.einsum('bnc,bpc->bnp', unknown, known)
    d = jnp.clip(d, 0.0, None)
    negd, ind = jax.lax.top_k(-d, 3)
    return jax.lax.stop_gradient(-negd), ind


def _linear_block(x, lin_params):
    for p in lin_params:
        x = x @ p['W'].T + p['b']
        x = _bn_train(x, p['g'], p['be'], (0,))
        x = jax.nn.relu(x)
    return x


def _feature_prop(xyz1, xyz2, feat1, feat2, fp_params):
    dist2, ind = _three_nn(jnp.transpose(xyz2, (0, 2, 1)), jnp.transpose(xyz1, (0, 2, 1)))
    inv = 1.0 / (dist2 + 1e-10)
    weights = inv / jnp.sum(inv, axis=2, keepdims=True)
    new = jnp.sum(_group_gather(feat1, ind) * weights[:, None, :, :], axis=3)
    new = jnp.concatenate([new, feat2], axis=1)
    x = new[..., None]
    for p in fp_params:
        x = jnp.einsum('bcns,oc->bons', x, p['W']) + p['b'][None, :, None, None]
        x = _bn_train(x, p['g'], p['be'], (0, 2, 3))
        x = jax.nn.relu(x)
    return x[..., 0]


def _fps_body(xyz_ref, idx_ref, dists_ref):
    B, _, NS, NL = xyz_ref.shape
    N = NS * NL
    x0 = xyz_ref[:, 0]
    x1 = xyz_ref[:, 1]
    x2 = xyz_ref[:, 2]
    i_s = jax.lax.broadcasted_iota(jnp.int32, (B, NS, NL), 1)
    i_l = jax.lax.broadcasted_iota(jnp.int32, (B, NS, NL), 2)
    n_idx = i_s * NL + i_l
    dists_ref[...] = jnp.full((B, NS, NL), 1e10, jnp.float32)

    def body(i, state):
        far, idxacc = state
        idxacc = jnp.where(n_idx == i, far, idxacc)
        onehot = n_idx == far
        c0 = jnp.sum(jnp.where(onehot, x0, 0.0), axis=(1, 2), keepdims=True)
        c1 = jnp.sum(jnp.where(onehot, x1, 0.0), axis=(1, 2), keepdims=True)
        c2 = jnp.sum(jnp.where(onehot, x2, 0.0), axis=(1, 2), keepdims=True)
        d = (x0 - c0) ** 2 + (x1 - c1) ** 2 + (x2 - c2) ** 2
        dists = jnp.minimum(dists_ref[...], d)
        dists_ref[...] = dists
        m = jnp.max(dists, axis=(1, 2), keepdims=True)
        far_new = jnp.min(jnp.where(dists == m, n_idx, N), axis=(1, 2), keepdims=True)
        return far_new, idxacc

    init = (jnp.zeros((B, 1, 1), jnp.int32), jnp.zeros((B, NS, NL), jnp.int32))
    _, idxacc = jax.lax.fori_loop(0, N, body, init)
    idx_ref[...] = idxacc.reshape(B, N)


def _fps_pallas(xyz, interpret=False):
    B, C, N = xyz.shape
    NS, NL = 8, N // 8
    xyz_r = xyz.reshape(B, C, NS, NL)
    return pl.pallas_call(
        _fps_body,
        out_shape=jax.ShapeDtypeStruct((B, N), jnp.int32),
        scratch_shapes=[pltpu.VMEM((B, NS, NL), jnp.float32)],
        interpret=interpret,
    )(xyz_r)


def _final_conv_body(x_ref, w_ref, b_ref, o_ref):
    x = x_ref[0]
    w = w_ref[...]
    o_ref[0] = jnp.dot(w, x, preferred_element_type=jnp.float32) + b_ref[...][:, None]


def _final_conv(x, W, b):
    B, C, N = x.shape
    O = W.shape[0]
    return pl.pallas_call(
        _final_conv_body,
        grid=(B,),
        in_specs=[
            pl.BlockSpec((1, C, N), lambda i: (i, 0, 0)),
            pl.BlockSpec((O, C), lambda i: (0, 0)),
            pl.BlockSpec((O,), lambda i: (0,)),
        ],
        out_specs=pl.BlockSpec((1, O, N), lambda i: (i, 0, 0)),
        out_shape=jax.ShapeDtypeStruct((B, O, N), jnp.float32),
    )(x, W, b)


def _classifier_head(x, cls_params):
    for p in cls_params[:-1]:
        x = jnp.einsum('bcn,oc->bon', x, p['W']) + p['b'][None, :, None]
        x = _bn_train(x, p['g'], p['be'], (0, 2))
        x = jax.nn.relu(x)
    p = cls_params[-1]
    return _final_conv(x, p['W'], p['b'])


def kernel(xyz1, xyz2, feat1, feat2, params):
    xyz1_ind = _fps_pallas(xyz1)
    xyz1_1 = jax.vmap(lambda p, i: p[:, i])(xyz1, xyz1_ind)
    ind1 = _sample_k(xyz1, xyz1_1, N_SAMPLES)
    ind2 = _sample_k(xyz2, xyz1_1, N_SAMPLES)
    xyz1_g = _group_gather(xyz1, ind1) - xyz1_1[..., None]
    xyz2_g = _group_gather(xyz2, ind2) - xyz1_1[..., None]
    pc = params['pcconv']
    f1 = _pcconv(xyz1_g, feat1, ind1, pc[0])
    f2 = _pcconv(xyz2_g, feat2, ind2, pc[0])
    for i in range(1, N_LAYER - 1):
        f1 = f1 + _pcconv(xyz1_g, f1, ind1, pc[i])
        f2 = f2 + _pcconv(xyz2_g, f2, ind2, pc[i])
    f1 = _pcconv(xyz1_g, f1, ind1, pc[N_LAYER - 1])
    f2 = _pcconv(xyz2_g, f2, ind2, pc[N_LAYER - 1])
    p1 = _linear_block(jnp.max(f1, axis=2), params['lin'])[:, :, None]
    p2 = _linear_block(jnp.max(f2, axis=2), params['lin'])[:, :, None]
    P = f1.shape[-1]
    feat_final = jnp.concatenate([jnp.repeat(p1, P, axis=2), f1, jnp.repeat(p2, P, axis=2), f2], axis=1)
    feat_final = _feature_prop(xyz1_1, xyz1, feat_final, feat1, params['fp'])
    return _classifier_head(feat_final, params['cls'])


# speedup vs baseline: 3.7952x; 3.0996x over previous
import functools

import jax
import jax.numpy as jnp
from jax.experimental import pallas as pl
from jax.experimental.pallas import tpu as pltpu

N_POINTS = 2048
N_SAMPLES = 16
N_LAYER = 8


def _pdist2squared(x, y):
    xx = jnp.sum(x ** 2, axis=1)[:, :, None]
    yy = jnp.sum(y ** 2, axis=1)[:, None, :]
    d = xx + yy - 2.0 * jnp.einsum('bcn,bcp->bnp', x, y)
    d = jnp.where(jnp.isnan(d), 0.0, d)
    return jnp.clip(d, 0.0, None)


def _group_gather(feat, ind):
    return jax.vmap(lambda f, i: f[:, i])(feat, ind)


def _fps(xyz, npoint):
    B, N, _ = xyz.shape

    def body(i, state):
        idx, dists, far = state
        idx = idx.at[:, i].set(far)
        centroid = jnp.take_along_axis(xyz, far[:, None, None], axis=1)
        d = jnp.sum((xyz - centroid) ** 2, axis=-1)
        dists = jnp.minimum(dists, d)
        far = jnp.argmax(dists, axis=-1).astype(jnp.int32)
        return (idx, dists, far)

    init = (jnp.zeros((B, npoint), jnp.int32), jnp.full((B, N), 1e10, jnp.float32), jnp.zeros((B,), jnp.int32))
    idx, _, _ = jax.lax.fori_loop(0, npoint, body, init)
    return idx


def _bn_train(x, g, b, axes, eps=1e-3):
    m = jnp.mean(x, axis=axes, keepdims=True)
    v = jnp.var(x, axis=axes, keepdims=True)
    shape = [1] * x.ndim
    shape[1] = -1
    return (x - m) / jnp.sqrt(v + eps) * g.reshape(shape) + b.reshape(shape)


def _sample_k(xyz_full, xyz_sampled, num_samples):
    dist = _pdist2squared(xyz_full, xyz_sampled)
    _, ind = jax.lax.top_k(-jnp.transpose(dist, (0, 2, 1)), num_samples)
    return ind


def _pcconv(xyz_grouped, feat, ind, layer_params):
    feat_g = _group_gather(feat, ind)
    w = xyz_grouped
    for p in layer_params:
        w = jnp.einsum('bcps,oc->bops', w, p['W']) + p['b'][None, :, None, None]
        w = _bn_train(w, p['g'], p['be'], (0, 2, 3))
        w = jax.nn.relu(w)
    return jnp.einsum('bcps,bwps->bwp', feat_g, w) / w.shape[3]


def _three_nn(unknown, known):
    d = jnp.sum(unknown ** 2, -1)[:, :, None] + jnp.sum(known ** 2, -1)[:, None, :] - 2.0 * jnp.einsum('bnc,bpc->bnp', unknown, known)
    d = jnp.clip(d, 0.0, None)
    negd, ind = jax.lax.top_k(-d, 3)
    return jax.lax.stop_gradient(-negd), ind


def _linear_block(x, lin_params):
    for p in lin_params:
        x = x @ p['W'].T + p['b']
        x = _bn_train(x, p['g'], p['be'], (0,))
        x = jax.nn.relu(x)
    return x


def _feature_prop(xyz1, xyz2, feat1, feat2, fp_params):
    dist2, ind = _three_nn(jnp.transpose(xyz2, (0, 2, 1)), jnp.transpose(xyz1, (0, 2, 1)))
    inv = 1.0 / (dist2 + 1e-10)
    weights = inv / jnp.sum(inv, axis=2, keepdims=True)
    new = jnp.sum(_group_gather(feat1, ind) * weights[:, None, :, :], axis=3)
    new = jnp.concatenate([new, feat2], axis=1)
    x = new[..., None]
    for p in fp_params:
        x = jnp.einsum('bcns,oc->bons', x, p['W']) + p['b'][None, :, None, None]
        x = _bn_train(x, p['g'], p['be'], (0, 2, 3))
        x = jax.nn.relu(x)
    return x[..., 0]


def _fps_body(xyz_ref, idx_ref, dists_ref):
    B, _, NS, NL = xyz_ref.shape
    N = NS * NL
    x0 = xyz_ref[:, 0]
    x1 = xyz_ref[:, 1]
    x2 = xyz_ref[:, 2]
    i_s = jax.lax.broadcasted_iota(jnp.int32, (B, NS, NL), 1)
    i_l = jax.lax.broadcasted_iota(jnp.int32, (B, NS, NL), 2)
    n_idx = i_s * NL + i_l
    dists_ref[...] = jnp.full((B, NS, NL), 1e10, jnp.float32)

    def body(i, state):
        far, idxacc = state
        idxacc = jnp.where(n_idx == i, far, idxacc)
        onehot = n_idx == far
        c0 = jnp.sum(jnp.where(onehot, x0, 0.0), axis=(1, 2), keepdims=True)
        c1 = jnp.sum(jnp.where(onehot, x1, 0.0), axis=(1, 2), keepdims=True)
        c2 = jnp.sum(jnp.where(onehot, x2, 0.0), axis=(1, 2), keepdims=True)
        d = (x0 - c0) ** 2 + (x1 - c1) ** 2 + (x2 - c2) ** 2
        dists = jnp.minimum(dists_ref[...], d)
        dists_ref[...] = dists
        m = jnp.max(dists, axis=(1, 2), keepdims=True)
        far_new = jnp.min(jnp.where(dists == m, n_idx, N), axis=(1, 2), keepdims=True)
        return far_new, idxacc

    init = (jnp.zeros((B, 1, 1), jnp.int32), jnp.zeros((B, NS, NL), jnp.int32))
    _, idxacc = jax.lax.fori_loop(0, N, body, init)
    idx_ref[...] = idxacc.reshape(B, N)


def _fps_pallas(xyz, interpret=False):
    B, C, N = xyz.shape
    NS, NL = 8, N // 8
    xyz_r = xyz.reshape(B, C, NS, NL)
    return pl.pallas_call(
        _fps_body,
        out_shape=jax.ShapeDtypeStruct((B, N), jnp.int32),
        scratch_shapes=[pltpu.VMEM((B, NS, NL), jnp.float32)],
        interpret=interpret,
    )(xyz_r)


def _sc_gather_flat(table, idx2d):
    from jax.experimental.pallas import tpu_sc as plsc
    NROWS = idx2d.shape[0]
    NW = 32
    rpw = NROWS // NW
    assert rpw * NW == NROWS
    mesh = plsc.VectorSubcoreMesh(core_axis_name="c", subcore_axis_name="s")

    @functools.partial(
        pl.kernel, mesh=mesh,
        out_type=jax.ShapeDtypeStruct((NROWS, 128), jnp.float32),
        scratch_types=[
            pltpu.VMEM((rpw, 128), jnp.int32),
            pltpu.VMEM((rpw, 128), jnp.float32),
            pltpu.SemaphoreType.DMA,
        ],
    )
    def k(tab_hbm, idx_hbm, out_hbm, idx_v, rows_v, sem):
        wid = jax.lax.axis_index("s") * 2 + jax.lax.axis_index("c")
        base = wid * rpw
        pltpu.sync_copy(idx_hbm.at[pl.ds(base, rpw)], idx_v)

        def body(c, carry):
            pltpu.async_copy(tab_hbm.at[idx_v.at[c]], rows_v.at[c], sem).wait()
            return carry

        jax.lax.fori_loop(0, rpw, body, 0)
        pltpu.sync_copy(rows_v, out_hbm.at[pl.ds(base, rpw)])

    return k(table, idx2d)


def _y1stats_body(x_ref, a1_ref, c1_ref, m1_ref, s_ref, *, centered):
    bc = pl.program_id(0)
    t = pl.program_id(1)
    xs = x_ref[0]
    y1 = jnp.dot(a1_ref[...], xs, preferred_element_type=jnp.float32) + c1_ref[...]

    @pl.when(jnp.logical_and(bc % 4 == 0, t == 0))
    def _():
        s_ref[0] = jnp.zeros_like(s_ref[0])

    if centered:
        d = y1 - m1_ref[0]
        s_ref[0] += jnp.sum(d * d, axis=1, keepdims=True)
    else:
        s_ref[0] += jnp.sum(y1, axis=1, keepdims=True)


def _y1stats(x, A1, c1, m1, centered, n_tiles=4):
    R, C, M = x.shape
    K = A1.shape[0]
    T = M // n_tiles
    return pl.pallas_call(
        functools.partial(_y1stats_body, centered=centered),
        grid=(R, n_tiles),
        in_specs=[
            pl.BlockSpec((1, C, T), lambda bc, t: (bc, 0, t)),
            pl.BlockSpec((K, C), lambda bc, t: (0, 0)),
            pl.BlockSpec((K, 1), lambda bc, t: (0, 0)),
            pl.BlockSpec((1, K, 1), lambda bc, t: (bc // 4, 0, 0)),
        ],
        out_specs=pl.BlockSpec((1, K, 1), lambda bc, t: (bc // 4, 0, 0)),
        out_shape=jax.ShapeDtypeStruct((2, K, 1), jnp.float32),
    )(x, A1, c1, m1)


def _blockdiag_y2(z1, w2a_ref, w2b_ref, c2_ref):
    c2 = c2_ref[...]
    parts = []
    for l in range(7):
        zl = z1[l * 16:(l + 1) * 16]
        parts.append(jnp.dot(w2a_ref[l], zl, preferred_element_type=jnp.float32)
                     + c2[l * 32:(l + 1) * 32])
    parts.append(jnp.dot(w2b_ref[...], z1[112:176], preferred_element_type=jnp.float32)
                 + c2[224:352])
    return jnp.concatenate(parts, axis=0)


def _bn_apply(y, bnp):
    m = bnp[0][:, None]
    sd = bnp[1][:, None]
    g = bnp[2][:, None]
    be = bnp[3][:, None]
    return jnp.maximum((y - m) / sd * g + be, 0.0)


def _y2stats_body(x_ref, a1_ref, c1_ref, bn1_ref, w2a_ref, w2b_ref, c2_ref,
                  m2_ref, s_ref, *, centered):
    bc = pl.program_id(0)
    t = pl.program_id(1)
    xs = x_ref[0]
    y1 = jnp.dot(a1_ref[...], xs, preferred_element_type=jnp.float32) + c1_ref[...]
    z1 = _bn_apply(y1, bn1_ref[0])
    y2 = _blockdiag_y2(z1, w2a_ref, w2b_ref, c2_ref)

    @pl.when(jnp.logical_and(bc % 4 == 0, t == 0))
    def _():
        s_ref[0] = jnp.zeros_like(s_ref[0])

    if centered:
        d = y2 - m2_ref[0]
        s_ref[0] += jnp.sum(d * d, axis=1, keepdims=True)
    else:
        s_ref[0] += jnp.sum(y2, axis=1, keepdims=True)


def _y2stats(x, A1, c1, bn1, W2a, W2b, c2, m2, centered, n_tiles=4):
    R, C, M = x.shape
    K1 = A1.shape[0]
    K2 = c2.shape[0]
    T = M // n_tiles
    return pl.pallas_call(
        functools.partial(_y2stats_body, centered=centered),
        grid=(R, n_tiles),
        in_specs=[
            pl.BlockSpec((1, C, T), lambda bc, t: (bc, 0, t)),
            pl.BlockSpec((K1, C), lambda bc, t: (0, 0)),
            pl.BlockSpec((K1, 1), lambda bc, t: (0, 0)),
            pl.BlockSpec((1, 4, K1), lambda bc, t: (bc // 4, 0, 0)),
            pl.BlockSpec(W2a.shape, lambda bc, t: (0, 0, 0)),
            pl.BlockSpec(W2b.shape, lambda bc, t: (0, 0)),
            pl.BlockSpec((K2, 1), lambda bc, t: (0, 0)),
            pl.BlockSpec((1, K2, 1), lambda bc, t: (bc // 4, 0, 0)),
        ],
        out_specs=pl.BlockSpec((1, K2, 1), lambda bc, t: (bc // 4, 0, 0)),
        out_shape=jax.ShapeDtypeStruct((2, K2, 1), jnp.float32),
    )(x, A1, c1, bn1, W2a, W2b, c2, m2)


def _pcstep_body(x_ref, g_ref, f_ref, a1_ref, c1_ref, bn1_ref, w2_ref, c2_ref,
                 bn2_ref, fo_ref, cs_ref):
    S = x_ref.shape[2]
    a1 = a1_ref[...]
    c1 = c1_ref[...]
    w2 = w2_ref[...]
    c2 = c2_ref[...]
    bn1 = bn1_ref[0]
    bn2 = bn2_ref[0]
    C2 = w2.shape[0]
    P = x_ref.shape[3]
    delta = jnp.zeros((C2, P), jnp.float32)
    for s in range(S):
        xs = x_ref[0, :, s, :]
        y1 = jnp.dot(a1, xs, preferred_element_type=jnp.float32) + c1
        z1 = _bn_apply(y1, bn1)
        y2 = jnp.dot(w2, z1, preferred_element_type=jnp.float32) + c2
        z2 = _bn_apply(y2, bn2)
        delta = delta + z2 * g_ref[0, 0, s, :][None, :]
    fn = f_ref[0] + delta * (1.0 / S)
    fo_ref[0] = fn
    cs_ref[0] = jnp.sum(fn, axis=0, keepdims=True)


def _pcconv_step(x_sm, g_sm, f_prev, A1, c1, bn1, W2, c2, bn2):
    R, _, S, P = x_sm.shape
    C1 = A1.shape[0]
    C2 = W2.shape[0]
    return pl.pallas_call(
        _pcstep_body,
        grid=(R,),
        in_specs=[
            pl.BlockSpec((1, 3, S, P), lambda bc: (bc, 0, 0, 0)),
            pl.BlockSpec((1, 1, S, P), lambda bc: (bc, 0, 0, 0)),
            pl.BlockSpec((1, C2, P), lambda bc: (bc, 0, 0)),
            pl.BlockSpec((C1, 3), lambda bc: (0, 0)),
            pl.BlockSpec((C1, 1), lambda bc: (0, 0)),
            pl.BlockSpec((1, 4, C1), lambda bc: (bc // 4, 0, 0)),
            pl.BlockSpec((C2, C1), lambda bc: (0, 0)),
            pl.BlockSpec((C2, 1), lambda bc: (0, 0)),
            pl.BlockSpec((1, 4, C2), lambda bc: (bc // 4, 0, 0)),
        ],
        out_specs=[
            pl.BlockSpec((1, C2, P), lambda bc: (bc, 0, 0)),
            pl.BlockSpec((1, 1, P), lambda bc: (bc, 0, 0)),
        ],
        out_shape=[
            jax.ShapeDtypeStruct((R, C2, P), jnp.float32),
            jax.ShapeDtypeStruct((R, 1, P), jnp.float32),
        ],
    )(x_sm, g_sm, f_prev, A1, c1, bn1, W2, c2, bn2)


def _pcconv_stack(xyz_g_sm, idxT_flat, cs0, pc_params):
    R, _, S, P = xyz_g_sm.shape
    M = S * P
    x_flat = xyz_g_sm.reshape(R, 3, M)
    N4 = jnp.float32(4 * M)

    A1cat = jnp.concatenate([pc_params[l][0]['W'] for l in range(N_LAYER)], 0)
    c1cat = jnp.concatenate([pc_params[l][0]['b'] for l in range(N_LAYER)])[:, None]
    g1cat = jnp.concatenate([pc_params[l][0]['g'] for l in range(N_LAYER)])
    be1cat = jnp.concatenate([pc_params[l][0]['be'] for l in range(N_LAYER)])
    W2a = jnp.stack([pc_params[l][1]['W'] for l in range(N_LAYER - 1)])
    W2b = pc_params[N_LAYER - 1][1]['W']
    c2cat = jnp.concatenate([pc_params[l][1]['b'] for l in range(N_LAYER)])[:, None]
    g2cat = jnp.concatenate([pc_params[l][1]['g'] for l in range(N_LAYER)])
    be2cat = jnp.concatenate([pc_params[l][1]['be'] for l in range(N_LAYER)])

    K1 = A1cat.shape[0]
    K2 = c2cat.shape[0]
    zero1 = jnp.zeros((2, K1, 1), jnp.float32)
    zero2 = jnp.zeros((2, K2, 1), jnp.float32)

    m1 = _y1stats(x_flat, A1cat, c1cat, zero1, False) / N4
    v1 = _y1stats(x_flat, A1cat, c1cat, m1, True) / N4
    sd1 = jnp.sqrt(v1 + 1e-3)
    bn1 = jnp.stack([m1[:, :, 0], sd1[:, :, 0],
                     jnp.tile(g1cat, (2, 1)), jnp.tile(be1cat, (2, 1))], axis=1)

    m2 = _y2stats(x_flat, A1cat, c1cat, bn1, W2a, W2b, c2cat, zero2, False) / N4
    v2 = _y2stats(x_flat, A1cat, c1cat, bn1, W2a, W2b, c2cat, m2, True) / N4
    sd2 = jnp.sqrt(v2 + 1e-3)
    bn2 = jnp.stack([m2[:, :, 0], sd2[:, :, 0],
                     jnp.tile(g2cat, (2, 1)), jnp.tile(be2cat, (2, 1))], axis=1)

    J = M
    V = cs0.shape[1]
    row_off = (jnp.arange(R, dtype=jnp.int32) * V)[:, None]
    gidx2d = (idxT_flat + row_off).reshape(R * J // 128, 128)
    cs = cs0
    f = None
    for l in range(N_LAYER):
        C1 = pc_params[l][0]['W'].shape[0]
        C2 = pc_params[l][1]['W'].shape[0]
        o1 = 16 * l if l < N_LAYER - 1 else 112
        o2 = 32 * l if l < N_LAYER - 1 else 224
        g_flat = _sc_gather_flat(cs.reshape(-1), gidx2d)
        g_sm = g_flat.reshape(R, 1, S, P)
        if l == 0 or l == N_LAYER - 1:
            f_prev = jnp.zeros((R, C2, P), jnp.float32)
        else:
            f_prev = f
        f, cs3 = _pcconv_step(
            xyz_g_sm, g_sm, f_prev,
            A1cat[o1:o1 + C1], c1cat[o1:o1 + C1], bn1[:, :, o1:o1 + C1],
            pc_params[l][1]['W'], c2cat[o2:o2 + C2], bn2[:, :, o2:o2 + C2])
        cs = cs3[:, 0, :]
    return f


def _final_conv_body(x_ref, w_ref, b_ref, o_ref):
    x = x_ref[0]
    w = w_ref[...]
    o_ref[0] = jnp.dot(w, x, preferred_element_type=jnp.float32) + b_ref[...][:, None]


def _final_conv(x, W, b):
    B, C, N = x.shape
    O = W.shape[0]
    return pl.pallas_call(
        _final_conv_body,
        grid=(B,),
        in_specs=[
            pl.BlockSpec((1, C, N), lambda i: (i, 0, 0)),
            pl.BlockSpec((O, C), lambda i: (0, 0)),
            pl.BlockSpec((O,), lambda i: (0,)),
        ],
        out_specs=pl.BlockSpec((1, O, N), lambda i: (i, 0, 0)),
        out_shape=jax.ShapeDtypeStruct((B, O, N), jnp.float32),
    )(x, W, b)


def _classifier_head(x, cls_params):
    for p in cls_params[:-1]:
        x = jnp.einsum('bcn,oc->bon', x, p['W']) + p['b'][None, :, None]
        x = _bn_train(x, p['g'], p['be'], (0, 2))
        x = jax.nn.relu(x)
    p = cls_params[-1]
    return _final_conv(x, p['W'], p['b'])


def kernel(xyz1, xyz2, feat1, feat2, params):
    xyz1_ind = _fps_pallas(xyz1)
    xyz1_1 = jax.vmap(lambda p, i: p[:, i])(xyz1, xyz1_ind)
    ind1 = _sample_k(xyz1, xyz1_1, N_SAMPLES)
    ind2 = _sample_k(xyz2, xyz1_1, N_SAMPLES)
    pc = params['pcconv']
    ind1T = jnp.transpose(ind1, (0, 2, 1))
    ind2T = jnp.transpose(ind2, (0, 2, 1))
    xyz1_g_sm = jax.vmap(lambda p, i: p[:, i])(xyz1, ind1T) - xyz1_1[:, :, None, :]
    xyz2_g_sm = jax.vmap(lambda p, i: p[:, i])(xyz2, ind2T) - xyz1_1[:, :, None, :]
    x_sm = jnp.concatenate([xyz1_g_sm, xyz2_g_sm], 0)
    B = x_sm.shape[0] // 2
    idxT = jnp.concatenate([ind1T, ind2T], 0).reshape(2 * B, -1).astype(jnp.int32)
    cs0 = jnp.concatenate([feat1[:, 0, :], feat2[:, 0, :]], 0)
    fout = _pcconv_stack(x_sm, idxT, cs0, pc)
    f1, f2 = fout[:B], fout[B:]
    p1 = _linear_block(jnp.max(f1, axis=2), params['lin'])[:, :, None]
    p2 = _linear_block(jnp.max(f2, axis=2), params['lin'])[:, :, None]
    P = f1.shape[-1]
    feat_final = jnp.concatenate([jnp.repeat(p1, P, axis=2), f1, jnp.repeat(p2, P, axis=2), f2], axis=1)
    feat_final = _feature_prop(xyz1_1, xyz1, feat_final, feat1, params['fp'])
    return _classifier_head(feat_final, params['cls'])
